# Initial kernel scaffold; baseline (speedup 1.0000x reference)
import jax
import jax.numpy as jnp
from jax.experimental import pallas as pl
from jax.experimental.pallas import tpu as pltpu

H = 50
NBASIS = 8
LAYERS = 6
BAR_HALF = 0.1
MAX_RADIUS = 0.06


def _fused_row_kernel(y_ref, We_ref, W1_ref, W2_ref, W3_ref, bm_ref, Wu_ref,
                      bu_ref, Wo_ref, bo_ref, out_ref):
    R = y_ref.shape[1]
    ca = y_ref[0][:, 0:3]
    prev = jnp.concatenate([ca[0:1], ca[:-1]], axis=0)

    a1 = ca[:, 2:3]
    dx = BAR_HALF * jnp.cos(a1)
    dy = BAR_HALF * jnp.sin(a1)
    ex = ca[:, 0:1] - prev[:, 0:1]
    ey = ca[:, 1:2] - prev[:, 1:2]
    elen6 = jnp.sqrt(ex * ex + ey * ey + 1e-12)
    elen2 = jnp.sqrt(dx * dx + dy * dy + 1e-12)

    centers = jnp.linspace(0.0, MAX_RADIUS, NBASIS).astype(jnp.float32)
    inv = jnp.float32(NBASIS / MAX_RADIUS)
    basis6 = jnp.exp(-jnp.square((elen6 - centers[None, :]) * inv))
    basis2 = jnp.exp(-jnp.square((elen2 - centers[None, :]) * inv))

    one = jnp.ones((R, 1), jnp.float32)
    zero = jnp.zeros((R, 1), jnp.float32)
    ef6 = jnp.concatenate([one, zero, basis6, ex, ey, zero], axis=1)
    ef2 = jnp.concatenate([zero, one, basis2, dx, dy, zero], axis=1)
    ef3 = jnp.concatenate([zero, one, basis2, -dx, -dy, zero], axis=1)

    We = We_ref[...]
    hc = We[0:1] + We[1:2]
    he = We[0:1] + We[2:3]
    zR = jnp.zeros((R, H), jnp.float32)
    h1 = hc
    h4 = hc + zR
    h3 = he + zR
    h5 = he + zR

    def dot(a, b):
        return jnp.dot(a, b, preferred_element_type=jnp.float32)

    g = jax.nn.gelu
    for l in range(LAYERS):
        W1 = W1_ref[l]
        W2 = W2_ref[l]
        W3 = W3_ref[l]
        bm = bm_ref[l]
        Wu = Wu_ref[l]
        bu = bu_ref[l]
        m6 = g(dot(h4, W2) + dot(ef6, W3) + (dot(h1, W1) + bm))
        s4 = dot(h4, W1)
        m2 = g(s4 + dot(h3, W2) + dot(ef2, W3) + bm)
        m3 = g(s4 + dot(h5, W2) + dot(ef3, W3) + bm)
        h4 = g(dot(h4 + m6, Wu) + bu)
        h3 = g(dot(h3 + m2, Wu) + bu)
        h5 = g(dot(h5 + m3, Wu) + bu)
        h1 = g(dot(h1, Wu) + bu)

    Wo = Wo_ref[...]
    bo = bo_ref[...]
    o3 = dot(h3, Wo) + bo
    o4 = dot(h4, Wo) + bo
    o5 = dot(h5, Wo) + bo

    fx = o3[:, 0:1] + o4[:, 0:1] + o5[:, 0:1]
    fy = o3[:, 1:2] + o4[:, 1:2] + o5[:, 1:2]
    tq = dx * (o3[:, 1:2] - o5[:, 1:2]) - dy * (o3[:, 0:1] - o5[:, 0:1])
    res = jnp.concatenate([fx, fy, tq], axis=1)

    row = jax.lax.broadcasted_iota(jnp.int32, (R, 1), 0)
    mask = (row >= 1) & (row <= R - 2)
    out_ref[0] = jnp.where(mask, res, 0.0)


def kernel(y, W_embed, W_msg, b_msg, W_upd, b_upd, W_out, b_out):
    B, L, _ = y.shape
    W1 = W_msg[:, 0:H, :]
    W2 = W_msg[:, H:2 * H, :]
    W3 = W_msg[:, 2 * H:, :]
    bm = b_msg.reshape(LAYERS, 1, H)
    bu = b_upd.reshape(LAYERS, 1, H)
    bo = b_out.reshape(1, 3)

    def full(a):
        return pl.BlockSpec(a.shape, lambda b, _n=a.ndim: (0,) * _n)

    out = pl.pallas_call(
        _fused_row_kernel,
        grid=(B,),
        in_specs=[
            pl.BlockSpec((1, L, 6), lambda b: (b, 0, 0)),
            full(W_embed), full(W1), full(W2), full(W3), full(bm),
            full(W_upd), full(bu), full(W_out), full(bo),
        ],
        out_specs=pl.BlockSpec((1, L, 3), lambda b: (b, 0, 0)),
        out_shape=jax.ShapeDtypeStruct((B, L, 3), jnp.float32),
        compiler_params=pltpu.CompilerParams(
            dimension_semantics=("arbitrary",)),
    )(y, W_embed, W1, W2, W3, bm, W_upd, bu, W_out, bo)
    return out


# speedup vs baseline: 55.2683x; 55.2683x over previous
import jax
import jax.numpy as jnp
import numpy as np
from jax.experimental import pallas as pl
from jax.experimental.pallas import tpu as pltpu

H = 50
NBASIS = 8
LAYERS = 6
BAR_HALF = 0.1
MAX_RADIUS = 0.06


def _fused_row_kernel(y_ref, We_ref, W1_ref, W2_ref, W3_ref, bm_ref, Wu_ref,
                      bu_ref, Wo_ref, bo_ref, out_ref):
    R = y_ref.shape[1]
    ca = y_ref[0][:, 0:3]
    prev = jnp.concatenate([ca[0:1], ca[:-1]], axis=0)

    a1 = ca[:, 2:3]
    dx = BAR_HALF * jnp.cos(a1)
    dy = BAR_HALF * jnp.sin(a1)
    ex = ca[:, 0:1] - prev[:, 0:1]
    ey = ca[:, 1:2] - prev[:, 1:2]
    elen6 = jnp.sqrt(ex * ex + ey * ey + 1e-12)
    elen2 = jnp.sqrt(dx * dx + dy * dy + 1e-12)

    centers = [float(c) for c in
               np.linspace(0.0, MAX_RADIUS, NBASIS).astype(np.float32)]
    inv = float(np.float32(NBASIS / MAX_RADIUS))
    basis6 = jnp.concatenate(
        [jnp.exp(-jnp.square((elen6 - c) * inv)) for c in centers], axis=1)
    basis2 = jnp.concatenate(
        [jnp.exp(-jnp.square((elen2 - c) * inv)) for c in centers], axis=1)

    one = jnp.ones((R, 1), jnp.float32)
    zero = jnp.zeros((R, 1), jnp.float32)
    ef6 = jnp.concatenate([one, zero, basis6, ex, ey, zero], axis=1)
    ef2 = jnp.concatenate([zero, one, basis2, dx, dy, zero], axis=1)
    ef3 = jnp.concatenate([zero, one, basis2, -dx, -dy, zero], axis=1)

    We = We_ref[...]
    hc = We[0:1] + We[1:2]
    he = We[0:1] + We[2:3]
    zR = jnp.zeros((R, H), jnp.float32)
    h1 = hc
    h4 = hc + zR
    h3 = he + zR
    h5 = he + zR

    def dot(a, b):
        return jnp.dot(a, b, preferred_element_type=jnp.float32)

    g = jax.nn.gelu
    for l in range(LAYERS):
        W1 = W1_ref[l]
        W2 = W2_ref[l]
        W3 = W3_ref[l]
        bm = bm_ref[l]
        Wu = Wu_ref[l]
        bu = bu_ref[l]
        m6 = g(dot(h4, W2) + dot(ef6, W3) + (dot(h1, W1) + bm))
        s4 = dot(h4, W1)
        m2 = g(s4 + dot(h3, W2) + dot(ef2, W3) + bm)
        m3 = g(s4 + dot(h5, W2) + dot(ef3, W3) + bm)
        h4 = g(dot(h4 + m6, Wu) + bu)
        h3 = g(dot(h3 + m2, Wu) + bu)
        h5 = g(dot(h5 + m3, Wu) + bu)
        h1 = g(dot(h1, Wu) + bu)

    Wo = Wo_ref[...]
    bo = bo_ref[...]
    o3 = dot(h3, Wo) + bo
    o4 = dot(h4, Wo) + bo
    o5 = dot(h5, Wo) + bo

    fx = o3[:, 0:1] + o4[:, 0:1] + o5[:, 0:1]
    fy = o3[:, 1:2] + o4[:, 1:2] + o5[:, 1:2]
    tq = dx * (o3[:, 1:2] - o5[:, 1:2]) - dy * (o3[:, 0:1] - o5[:, 0:1])
    res = jnp.concatenate([fx, fy, tq], axis=1)

    row = jax.lax.broadcasted_iota(jnp.int32, (R, 1), 0)
    mask = (row >= 1) & (row <= R - 2)
    out_ref[0] = jnp.where(mask, res, 0.0)


def kernel(y, W_embed, W_msg, b_msg, W_upd, b_upd, W_out, b_out):
    B, L, _ = y.shape
    W1 = W_msg[:, 0:H, :]
    W2 = W_msg[:, H:2 * H, :]
    W3 = W_msg[:, 2 * H:, :]
    bm = b_msg.reshape(LAYERS, 1, H)
    bu = b_upd.reshape(LAYERS, 1, H)
    bo = b_out.reshape(1, 3)

    def full(a):
        return pl.BlockSpec(a.shape, lambda b, _n=a.ndim: (0,) * _n)

    out = pl.pallas_call(
        _fused_row_kernel,
        grid=(B,),
        in_specs=[
            pl.BlockSpec((1, L, 6), lambda b: (b, 0, 0)),
            full(W_embed), full(W1), full(W2), full(W3), full(bm),
            full(W_upd), full(bu), full(W_out), full(bo),
        ],
        out_specs=pl.BlockSpec((1, L, 3), lambda b: (b, 0, 0)),
        out_shape=jax.ShapeDtypeStruct((B, L, 3), jnp.float32),
        compiler_params=pltpu.CompilerParams(
            dimension_semantics=("arbitrary",)),
    )(y, W_embed, W1, W2, W3, bm, W_upd, bu, W_out, bo)
    return out


# speedup vs baseline: 159.9005x; 2.8932x over previous
import jax
import jax.numpy as jnp
import numpy as np
from jax.experimental import pallas as pl
from jax.experimental.pallas import tpu as pltpu

H = 50
HP = 56
NBASIS = 8
LAYERS = 6
BAR_HALF = 0.1
MAX_RADIUS = 0.06
GROWS = 2


def _fused_kernel(yt_ref, We_ref, Wm_ref, bm_ref, Wu_ref, bu_ref, Wo_ref,
                  bo_ref, out_ref):
    N = yt_ref.shape[1]
    L = N // GROWS
    cax = yt_ref[0:1, :]
    cay = yt_ref[1:2, :]
    a1 = yt_ref[2:3, :]

    prevx = jnp.concatenate([cax[:, 0:1], cax[:, :-1]], axis=1)
    prevy = jnp.concatenate([cay[:, 0:1], cay[:, :-1]], axis=1)

    dx = BAR_HALF * jnp.cos(a1)
    dy = BAR_HALF * jnp.sin(a1)
    ex = cax - prevx
    ey = cay - prevy
    elen6 = jnp.sqrt(ex * ex + ey * ey + 1e-12)
    elen2 = jnp.sqrt(dx * dx + dy * dy + 1e-12)

    centers = [float(c) for c in
               np.linspace(0.0, MAX_RADIUS, NBASIS).astype(np.float32)]
    inv = float(np.float32(NBASIS / MAX_RADIUS))
    basis6 = jnp.concatenate(
        [jnp.exp(-jnp.square((elen6 - c) * inv)) for c in centers], axis=0)
    basis2 = jnp.concatenate(
        [jnp.exp(-jnp.square((elen2 - c) * inv)) for c in centers], axis=0)

    one = jnp.ones((1, N), jnp.float32)
    zero = jnp.zeros((1, N), jnp.float32)
    z3 = jnp.zeros((3, N), jnp.float32)
    ef6 = jnp.concatenate([one, zero, basis6, ex, ey, zero, z3], axis=0)
    ef2 = jnp.concatenate([zero, one, basis2, dx, dy, zero, z3], axis=0)
    ef3 = jnp.concatenate([zero, one, basis2, -dx, -dy, zero, z3], axis=0)
    EF = jnp.concatenate([ef2, ef6, ef3], axis=1)

    We = We_ref[...]
    hc = We[:, 0:1] + We[:, 1:2]
    he = We[:, 0:1] + We[:, 2:3]
    zN = jnp.zeros((HP, N), jnp.float32)
    S = jnp.concatenate([he + zN, hc + zN, he + zN], axis=1)
    h1 = hc
    h1b = hc + zN

    def dot(a, b):
        return jnp.dot(a, b, preferred_element_type=jnp.float32)

    g = jax.nn.gelu
    for l in range(LAYERS):
        Wm = Wm_ref[l]
        Wu = Wu_ref[l]
        bm = bm_ref[l]
        bu = bu_ref[l]
        H4 = S[:, N:2 * N]
        SRC = jnp.concatenate([H4, h1b, H4], axis=1)
        IN = jnp.concatenate([SRC, S, EF], axis=0)
        M = g(dot(Wm, IN) + bm)
        S = g(dot(Wu, S + M) + bu)
        h1 = g(dot(Wu, h1) + bu)
        h1b = h1 + zN

    O = dot(Wo_ref[...], S) + bo_ref[...]
    o3 = O[:, 0:N]
    o4 = O[:, N:2 * N]
    o5 = O[:, 2 * N:3 * N]
    fx = o3[0:1] + o4[0:1] + o5[0:1]
    fy = o3[1:2] + o4[1:2] + o5[1:2]
    tq = dx * (o3[1:2] - o5[1:2]) - dy * (o3[0:1] - o5[0:1])
    res = jnp.concatenate([fx, fy, tq], axis=0)

    lane = jax.lax.broadcasted_iota(jnp.int32, (1, N), 1) % L
    mask = (lane >= 1) & (lane <= L - 2)
    out_ref[...] = jnp.where(mask, res, 0.0)


def kernel(y, W_embed, W_msg, b_msg, W_upd, b_upd, W_out, b_out):
    B, L, _ = y.shape
    f32 = jnp.float32
    yt = y.transpose(2, 0, 1).reshape(6, B * L)

    z = lambda *s: jnp.zeros(s, f32)
    W1T = W_msg[:, 0:H, :].transpose(0, 2, 1)
    W2T = W_msg[:, H:2 * H, :].transpose(0, 2, 1)
    W3T = W_msg[:, 2 * H:, :].transpose(0, 2, 1)
    Wm = jnp.concatenate([
        jnp.concatenate([W1T, z(LAYERS, H, 6), W2T, z(LAYERS, H, 6),
                         W3T, z(LAYERS, H, 3)], axis=2),
        z(LAYERS, HP - H, 128)], axis=1)
    Wu = jnp.concatenate([
        jnp.concatenate([W_upd.transpose(0, 2, 1), z(LAYERS, H, HP - H)],
                        axis=2),
        z(LAYERS, HP - H, HP)], axis=1)
    bm = jnp.concatenate([b_msg, z(LAYERS, HP - H)], axis=1)[..., None]
    bu = jnp.concatenate([b_upd, z(LAYERS, HP - H)], axis=1)[..., None]
    We = jnp.concatenate([
        jnp.concatenate([W_embed.T, z(H, 5)], axis=1),
        z(HP - H, 8)], axis=0)
    Wo = jnp.concatenate([
        jnp.concatenate([W_out.T, z(3, HP - H)], axis=1),
        z(5, HP)], axis=0)
    bo = jnp.concatenate([b_out, z(5)]).reshape(8, 1)

    NL = GROWS * L

    def full(a):
        return pl.BlockSpec(a.shape, lambda g, _n=a.ndim: (0,) * _n)

    out = pl.pallas_call(
        _fused_kernel,
        grid=(B // GROWS,),
        in_specs=[
            pl.BlockSpec((6, NL), lambda g: (0, g)),
            full(We), full(Wm), full(bm), full(Wu), full(bu),
            full(Wo), full(bo),
        ],
        out_specs=pl.BlockSpec((3, NL), lambda g: (0, g)),
        out_shape=jax.ShapeDtypeStruct((3, B * L), f32),
        compiler_params=pltpu.CompilerParams(
            dimension_semantics=("arbitrary",)),
    )(yt, We, Wm, bm, Wu, bu, Wo, bo)
    return out.reshape(3, B, L).transpose(1, 2, 0)


# speedup vs baseline: 162.8587x; 1.0185x over previous
import jax
import jax.numpy as jnp
import numpy as np
from jax.experimental import pallas as pl
from jax.experimental.pallas import tpu as pltpu

H = 50
HP = 56
NBASIS = 8
LAYERS = 6
BAR_HALF = 0.1
MAX_RADIUS = 0.06
GROWS = 4


def _fused_kernel(yt_ref, We_ref, Wm_ref, bm_ref, Wu_ref, bu_ref, Wo_ref,
                  bo_ref, out_ref):
    N = yt_ref.shape[1]
    L = N // GROWS
    cax = yt_ref[0:1, :]
    cay = yt_ref[1:2, :]
    a1 = yt_ref[2:3, :]

    prevx = jnp.concatenate([cax[:, 0:1], cax[:, :-1]], axis=1)
    prevy = jnp.concatenate([cay[:, 0:1], cay[:, :-1]], axis=1)

    dx = BAR_HALF * jnp.cos(a1)
    dy = BAR_HALF * jnp.sin(a1)
    ex = cax - prevx
    ey = cay - prevy
    elen6 = jnp.sqrt(ex * ex + ey * ey + 1e-12)
    elen2 = jnp.sqrt(dx * dx + dy * dy + 1e-12)

    centers = [float(c) for c in
               np.linspace(0.0, MAX_RADIUS, NBASIS).astype(np.float32)]
    inv = float(np.float32(NBASIS / MAX_RADIUS))
    basis6 = jnp.concatenate(
        [jnp.exp(-jnp.square((elen6 - c) * inv)) for c in centers], axis=0)
    basis2 = jnp.concatenate(
        [jnp.exp(-jnp.square((elen2 - c) * inv)) for c in centers], axis=0)

    one = jnp.ones((1, N), jnp.float32)
    zero = jnp.zeros((1, N), jnp.float32)
    z3 = jnp.zeros((3, N), jnp.float32)
    ef6 = jnp.concatenate([one, zero, basis6, ex, ey, zero, z3], axis=0)
    ef2 = jnp.concatenate([zero, one, basis2, dx, dy, zero, z3], axis=0)
    ef3 = jnp.concatenate([zero, one, basis2, -dx, -dy, zero, z3], axis=0)
    EF = jnp.concatenate([ef2, ef6, ef3], axis=1)

    We = We_ref[...]
    hc = We[:, 0:1] + We[:, 1:2]
    he = We[:, 0:1] + We[:, 2:3]
    zN = jnp.zeros((HP, N), jnp.float32)
    S = jnp.concatenate([he + zN, hc + zN, he + zN], axis=1)
    h1 = hc
    h1b = hc + zN

    def dot(a, b):
        return jnp.dot(a, b, preferred_element_type=jnp.float32)

    g = jax.nn.gelu
    for l in range(LAYERS):
        Wm = Wm_ref[l]
        Wu = Wu_ref[l]
        bm = bm_ref[l]
        bu = bu_ref[l]
        H4 = S[:, N:2 * N]
        SRC = jnp.concatenate([H4, h1b, H4], axis=1)
        IN = jnp.concatenate([SRC, S, EF], axis=0)
        M = g(dot(Wm, IN) + bm)
        S = g(dot(Wu, S + M) + bu)
        h1 = g(dot(Wu, h1) + bu)
        h1b = h1 + zN

    O = dot(Wo_ref[...], S) + bo_ref[...]
    o3 = O[:, 0:N]
    o4 = O[:, N:2 * N]
    o5 = O[:, 2 * N:3 * N]
    fx = o3[0:1] + o4[0:1] + o5[0:1]
    fy = o3[1:2] + o4[1:2] + o5[1:2]
    tq = dx * (o3[1:2] - o5[1:2]) - dy * (o3[0:1] - o5[0:1])
    res = jnp.concatenate([fx, fy, tq], axis=0)

    lane = jax.lax.broadcasted_iota(jnp.int32, (1, N), 1) % L
    mask = (lane >= 1) & (lane <= L - 2)
    out_ref[...] = jnp.where(mask, res, 0.0)


def kernel(y, W_embed, W_msg, b_msg, W_upd, b_upd, W_out, b_out):
    B, L, _ = y.shape
    f32 = jnp.float32
    yt = y.transpose(2, 0, 1).reshape(6, B * L)

    z = lambda *s: jnp.zeros(s, f32)
    W1T = W_msg[:, 0:H, :].transpose(0, 2, 1)
    W2T = W_msg[:, H:2 * H, :].transpose(0, 2, 1)
    W3T = W_msg[:, 2 * H:, :].transpose(0, 2, 1)
    Wm = jnp.concatenate([
        jnp.concatenate([W1T, z(LAYERS, H, 6), W2T, z(LAYERS, H, 6),
                         W3T, z(LAYERS, H, 3)], axis=2),
        z(LAYERS, HP - H, 128)], axis=1)
    Wu = jnp.concatenate([
        jnp.concatenate([W_upd.transpose(0, 2, 1), z(LAYERS, H, HP - H)],
                        axis=2),
        z(LAYERS, HP - H, HP)], axis=1)
    bm = jnp.concatenate([b_msg, z(LAYERS, HP - H)], axis=1)[..., None]
    bu = jnp.concatenate([b_upd, z(LAYERS, HP - H)], axis=1)[..., None]
    We = jnp.concatenate([
        jnp.concatenate([W_embed.T, z(H, 5)], axis=1),
        z(HP - H, 8)], axis=0)
    Wo = jnp.concatenate([
        jnp.concatenate([W_out.T, z(3, HP - H)], axis=1),
        z(5, HP)], axis=0)
    bo = jnp.concatenate([b_out, z(5)]).reshape(8, 1)

    NL = GROWS * L

    def full(a):
        return pl.BlockSpec(a.shape, lambda g, _n=a.ndim: (0,) * _n)

    out = pl.pallas_call(
        _fused_kernel,
        grid=(B // GROWS,),
        in_specs=[
            pl.BlockSpec((6, NL), lambda g: (0, g)),
            full(We), full(Wm), full(bm), full(Wu), full(bu),
            full(Wo), full(bo),
        ],
        out_specs=pl.BlockSpec((3, NL), lambda g: (0, g)),
        out_shape=jax.ShapeDtypeStruct((3, B * L), f32),
        compiler_params=pltpu.CompilerParams(
            dimension_semantics=("arbitrary",)),
    )(yt, We, Wm, bm, Wu, bu, Wo, bo)
    return out.reshape(3, B, L).transpose(1, 2, 0)


# speedup vs baseline: 171.0583x; 1.0503x over previous
import jax
import jax.numpy as jnp
import numpy as np
from jax.experimental import pallas as pl
from jax.experimental.pallas import tpu as pltpu

H = 50
HP = 56
NBASIS = 8
LAYERS = 6
BAR_HALF = 0.1
MAX_RADIUS = 0.06
GROWS = 4


def _fused_kernel(yt_ref, We_ref, Wm_ref, bm_ref, Wu_ref, bu_ref, Wo_ref,
                  bo_ref, out_ref, inb):
    N = yt_ref.shape[1]
    L = N // GROWS
    f32 = jnp.float32
    cax = yt_ref[0:1, :]
    cay = yt_ref[1:2, :]
    a1 = yt_ref[2:3, :]

    prevx = jnp.concatenate([cax[:, 0:1], cax[:, :-1]], axis=1)
    prevy = jnp.concatenate([cay[:, 0:1], cay[:, :-1]], axis=1)

    dx = BAR_HALF * jnp.cos(a1)
    dy = BAR_HALF * jnp.sin(a1)
    ex = cax - prevx
    ey = cay - prevy
    elen6 = jnp.sqrt(ex * ex + ey * ey + 1e-12)
    elen2 = jnp.sqrt(dx * dx + dy * dy + 1e-12)

    centers = [float(c) for c in
               np.linspace(0.0, MAX_RADIUS, NBASIS).astype(np.float32)]
    inv = float(np.float32(NBASIS / MAX_RADIUS))
    basis6 = jnp.concatenate(
        [jnp.exp(-jnp.square((elen6 - c) * inv)) for c in centers], axis=0)
    basis2 = jnp.concatenate(
        [jnp.exp(-jnp.square((elen2 - c) * inv)) for c in centers], axis=0)

    one = jnp.ones((1, N), f32)
    zero = jnp.zeros((1, N), f32)
    z2 = jnp.zeros((2, N), f32)
    ef6 = jnp.concatenate([one, zero, basis6, ex, ey, zero, one, z2], axis=0)
    ef2 = jnp.concatenate([zero, one, basis2, dx, dy, zero, one, z2], axis=0)
    ef3 = jnp.concatenate([zero, one, basis2, -dx, -dy, zero, one, z2],
                          axis=0)
    inb[112:128, 0:N] = ef2
    inb[112:128, N:2 * N] = ef6
    inb[112:128, 2 * N:3 * N] = ef3

    def zc(*s):
        return jnp.zeros(s, f32)

    WeT = jnp.transpose(We_ref[...])
    hc = jnp.concatenate([WeT[:, 0:1] + WeT[:, 1:2], zc(HP - H, 1)], axis=0)
    he = jnp.concatenate([WeT[:, 0:1] + WeT[:, 2:3], zc(HP - H, 1)], axis=0)

    Wmp = []
    Wup = []
    buc = []
    for l in range(LAYERS):
        Tl = jnp.transpose(jnp.concatenate(
            [Wm_ref[l], Wu_ref[l], bm_ref[l], bu_ref[l]], axis=0))
        wm = jnp.concatenate(
            [Tl[:, 0:H], zc(H, 6), Tl[:, H:2 * H], zc(H, 6),
             Tl[:, 2 * H:113], Tl[:, 163:164], zc(H, 2)], axis=1)
        Wmp.append(jnp.concatenate([wm, zc(HP - H, 128)], axis=0))
        wu = jnp.concatenate([Tl[:, 113:163], zc(H, 6)], axis=1)
        Wup.append(jnp.concatenate([wu, zc(HP - H, HP)], axis=0))
        buc.append(jnp.concatenate([Tl[:, 164:165], zc(HP - H, 1)], axis=0))

    WoT = jnp.transpose(Wo_ref[...])
    Wop = jnp.concatenate([
        jnp.concatenate([WoT, zc(3, HP - H)], axis=1), zc(5, HP)], axis=0)
    boc = jnp.concatenate([jnp.transpose(bo_ref[...]), zc(5, 1)], axis=0)

    zN = jnp.zeros((HP, N), f32)
    inb[0:56, 0:N] = hc + zN
    inb[0:56, N:2 * N] = hc + zN
    inb[0:56, 2 * N:3 * N] = hc + zN
    inb[56:112, 0:N] = he + zN
    inb[56:112, N:2 * N] = hc + zN
    inb[56:112, 2 * N:3 * N] = he + zN
    h1 = hc

    def dot(a, b):
        return jnp.dot(a, b, preferred_element_type=f32)

    g = jax.nn.gelu
    Snew = None
    for l in range(LAYERS):
        IN = inb[...]
        M = g(dot(Wmp[l], IN))
        S = IN[56:112, :]
        Snew = g(dot(Wup[l], S + M) + buc[l])
        if l < LAYERS - 1:
            h1 = g(dot(Wup[l], h1) + buc[l])
            inb[56:112, :] = Snew
            H4n = Snew[:, N:2 * N]
            inb[0:56, 0:N] = H4n
            inb[0:56, N:2 * N] = h1 + zN
            inb[0:56, 2 * N:3 * N] = H4n

    O = dot(Wop, Snew) + boc
    o3 = O[:, 0:N]
    o4 = O[:, N:2 * N]
    o5 = O[:, 2 * N:3 * N]
    fx = o3[0:1] + o4[0:1] + o5[0:1]
    fy = o3[1:2] + o4[1:2] + o5[1:2]
    tq = dx * (o3[1:2] - o5[1:2]) - dy * (o3[0:1] - o5[0:1])
    res = jnp.concatenate([fx, fy, tq], axis=0)

    lane = jax.lax.broadcasted_iota(jnp.int32, (1, N), 1) % L
    mask = (lane >= 1) & (lane <= L - 2)
    out_ref[...] = jnp.where(mask, res, 0.0)


def kernel(y, W_embed, W_msg, b_msg, W_upd, b_upd, W_out, b_out):
    B, L, _ = y.shape
    f32 = jnp.float32
    yt = y.transpose(2, 0, 1).reshape(6, B * L)
    NL = GROWS * L

    def full(a):
        return pl.BlockSpec(a.shape, lambda g, _n=a.ndim: (0,) * _n)

    bm3 = b_msg.reshape(LAYERS, 1, H)
    bu3 = b_upd.reshape(LAYERS, 1, H)
    bo2 = b_out.reshape(1, 3)

    out = pl.pallas_call(
        _fused_kernel,
        grid=(B // GROWS,),
        in_specs=[
            pl.BlockSpec((6, NL), lambda g: (0, g)),
            full(W_embed), full(W_msg), full(bm3), full(W_upd), full(bu3),
            full(W_out), full(bo2),
        ],
        out_specs=pl.BlockSpec((3, NL), lambda g: (0, g)),
        out_shape=jax.ShapeDtypeStruct((3, B * L), f32),
        scratch_shapes=[pltpu.VMEM((128, 3 * NL), f32)],
        compiler_params=pltpu.CompilerParams(
            dimension_semantics=("arbitrary",)),
    )(yt, W_embed, W_msg, bm3, W_upd, bu3, W_out, bo2)
    return out.reshape(3, B, L).transpose(1, 2, 0)
